# baseline (speedup 1.0000x reference)
import functools

import jax
import jax.numpy as jnp
from jax import lax
from jax.experimental import pallas as pl
from jax.experimental.pallas import tpu as pltpu
from jax.experimental.pallas import tpu_sc as plsc

N = 10000
E = 320000
F_IN = 128
H1 = 64
H2 = 32
C = 3
G = 64

NCORE = 2
NSUB = 16
K = 128
TCH = E // K
FCH = 138
SCH = 18
DCH = 78
NP = 10240
RPT = NP // NSUB


def _sc_mesh():
    return plsc.VectorSubcoreMesh(core_axis_name="c", subcore_axis_name="s")


def _zero_rows_buf(rows, h):
    def body(r, carry):
        for i in range(h // 16):
            rows[r, pl.ds(i * 16, 16)] = jnp.zeros((16,), jnp.float32)
        return carry
    lax.fori_loop(0, K, body, 0)


def _zero_acc_slice(rows, acc, base):
    nfull = RPT // K
    rem = RPT % K
    for b in range(nfull):
        pltpu.sync_copy(rows, acc.at[pl.ds(base + b * K, K)])
    if rem:
        pltpu.sync_copy(rows.at[pl.ds(0, rem)],
                        acc.at[pl.ds(base + nfull * K, rem)])


def _make_deg():
    @functools.partial(
        pl.kernel,
        out_type=jax.ShapeDtypeStruct((NCORE, NP, 16), jnp.float32),
        mesh=_sc_mesh(),
        scratch_types=[
            pltpu.VMEM((DCH + 1, K), jnp.int32),
            pltpu.VMEM((K, 16), jnp.float32),
            pltpu.VMEM_SHARED((NP, 16), jnp.float32),
            pltpu.SemaphoreType.DMA,
        ],
        compiler_params=pltpu.CompilerParams(use_tc_tiling_on_sc=False),
    )
    def deg(dst_hbm, out_hbm, didx, rows, acc, sem):
        c = lax.axis_index("c")
        s = lax.axis_index("s")
        w = s * NCORE + c
        base = s * RPT
        _zero_rows_buf(rows, 16)
        _zero_acc_slice(rows, acc, base)
        start = w * DCH + jnp.maximum(0, w - 28)
        count = DCH + jnp.where(w >= 28, 1, 0)
        pltpu.sync_copy(dst_hbm.at[pl.ds(start, DCH + 1)], didx)

        def fill_ones(r, carry):
            rows[r, :] = jnp.ones((16,), jnp.float32)
            return carry
        lax.fori_loop(0, K, fill_ones, 0)
        plsc.subcore_barrier()

        def chunk(j, carry):
            pltpu.async_copy(rows, acc.at[didx.at[j]], sem, add=True).wait()
            return carry
        lax.fori_loop(0, count, chunk, 0)
        plsc.subcore_barrier()
        pltpu.sync_copy(acc.at[pl.ds(base, RPT)],
                        out_hbm.at[c, pl.ds(base, RPT)])

    return deg


NBUF = 4


def _make_agg(h):
    @functools.partial(
        pl.kernel,
        out_type=jax.ShapeDtypeStruct((NCORE, NP, h), jnp.float32),
        mesh=_sc_mesh(),
        scratch_types=[
            pltpu.VMEM((FCH + 1, K), jnp.int32),
            pltpu.VMEM((FCH + 1, K), jnp.int32),
            [pltpu.VMEM((K, h), jnp.float32) for _ in range(NBUF)],
            pltpu.VMEM_SHARED((NP, h), jnp.float32),
            [pltpu.SemaphoreType.DMA for _ in range(NBUF)],
            pltpu.SemaphoreType.DMA,
        ],
        compiler_params=pltpu.CompilerParams(use_tc_tiling_on_sc=False),
    )
    def agg(table_hbm, src_hbm, dst_hbm, out_hbm, sidx, didx, rows, acc,
            sem_g, sem_s):
        c = lax.axis_index("c")
        s = lax.axis_index("s")
        base = s * RPT
        is_fast = c == 0
        start = jnp.where(is_fast, s * FCH,
                          16 * FCH + s * SCH + jnp.maximum(0, s - 12))
        count = jnp.where(is_fast, FCH, SCH + jnp.where(s >= 12, 1, 0))
        _zero_rows_buf(rows[0], h)
        _zero_acc_slice(rows[0], acc, base)

        @pl.when(is_fast)
        def _():
            pltpu.sync_copy(src_hbm.at[pl.ds(start, FCH)],
                            sidx.at[pl.ds(0, FCH)])
            pltpu.sync_copy(dst_hbm.at[pl.ds(start, FCH)],
                            didx.at[pl.ds(0, FCH)])

        @pl.when(jnp.logical_not(is_fast))
        def _():
            pltpu.sync_copy(src_hbm.at[pl.ds(start, SCH + 1)],
                            sidx.at[pl.ds(0, SCH + 1)])
            pltpu.sync_copy(dst_hbm.at[pl.ds(start, SCH + 1)],
                            didx.at[pl.ds(0, SCH + 1)])
        plsc.subcore_barrier()

        rounds = count // NBUF

        def round_body(i, carry):
            j0 = i * NBUF
            descs = [
                pltpu.async_copy(table_hbm.at[sidx.at[j0 + b]], rows[b],
                                 sem_g[b])
                for b in range(NBUF)
            ]
            prev = None
            for b in range(NBUF):
                descs[b].wait()
                if prev is not None:
                    prev.wait()
                prev = pltpu.async_copy(rows[b], acc.at[didx.at[j0 + b]],
                                        sem_s, add=True)
            prev.wait()
            return carry
        lax.fori_loop(0, rounds, round_body, 0)

        def tail_body(j, carry):
            pltpu.async_copy(table_hbm.at[sidx.at[j]], rows[0],
                             sem_g[0]).wait()
            pltpu.async_copy(rows[0], acc.at[didx.at[j]], sem_s,
                             add=True).wait()
            return carry
        lax.fori_loop(rounds * NBUF, count, tail_body, 0)
        plsc.subcore_barrier()
        pltpu.sync_copy(acc.at[pl.ds(base, RPT)],
                        out_hbm.at[c, pl.ds(base, RPT)])

    return agg


def _mm1(x, w1):
    def body(x_ref, w_ref, o_ref):
        o_ref[...] = jnp.dot(x_ref[...], w_ref[...],
                             preferred_element_type=jnp.float32)
    return pl.pallas_call(
        body, out_shape=jax.ShapeDtypeStruct((N, H1), jnp.float32))(x, w1)


def _scale(hw1, deg16):
    def body(hw_ref, deg_ref, hwp_ref, dinv_ref):
        deg = deg_ref[0, :N, 0] + deg_ref[1, :N, 0] + 1.0
        dinv = lax.rsqrt(deg)[:, None]
        dinv_ref[...] = dinv
        hwp_ref[...] = hw_ref[...] * dinv
    return pl.pallas_call(
        body,
        out_shape=[jax.ShapeDtypeStruct((N, H1), jnp.float32),
                   jax.ShapeDtypeStruct((N, 1), jnp.float32)])(hw1, deg16)


def _layer2(agg1, hw1p, dinv, b1, w2):
    def body(agg_ref, hwp_ref, dinv_ref, b_ref, w_ref, o_ref):
        aggsum = agg_ref[0, :N, :] + agg_ref[1, :N, :] + hwp_ref[...]
        hcur = jnp.maximum(aggsum * dinv_ref[...] + b_ref[...], 0.0)
        hw2 = jnp.dot(hcur, w_ref[...], preferred_element_type=jnp.float32)
        o_ref[...] = hw2 * dinv_ref[...]
    return pl.pallas_call(
        body, out_shape=jax.ShapeDtypeStruct((N, H2), jnp.float32))(
            agg1, hw1p, dinv, b1, w2)


def _final(agg2, hw2p, dinv, b2, batch2d, fc_w, fc_b):
    def body(agg_ref, hwp_ref, dinv_ref, b_ref, bat_ref, fw_ref, fb_ref,
             o_ref):
        aggsum = agg_ref[0, :N, :] + agg_ref[1, :N, :] + hwp_ref[...]
        hcur = jnp.maximum(aggsum * dinv_ref[...] + b_ref[...], 0.0)
        onehot = (bat_ref[...] == lax.broadcasted_iota(
            jnp.int32, (N, G), 1)).astype(jnp.float32)
        sums = lax.dot_general(onehot, hcur, (((0,), (0,)), ((), ())),
                               preferred_element_type=jnp.float32)
        cnt = jnp.sum(onehot, axis=0)[:, None]
        pooled = sums / jnp.maximum(cnt, 1.0)
        o_ref[...] = jnp.dot(pooled, fw_ref[...],
                             preferred_element_type=jnp.float32) + fb_ref[...]
    return pl.pallas_call(
        body, out_shape=jax.ShapeDtypeStruct((G, C), jnp.float32))(
            agg2, hw2p, dinv, b2, batch2d, fc_w, fc_b)


def kernel(x, edge_index, batch, W1, b1, W2, b2, fc_w, fc_b):
    src_r = edge_index[0].reshape(TCH, K)
    dst_r = edge_index[1].reshape(TCH, K)

    deg16 = _make_deg()(dst_r)
    hw1 = _mm1(x, W1)
    hw1p, dinv = _scale(hw1, deg16)
    agg1 = _make_agg(H1)(hw1p, src_r, dst_r)
    hw2p = _layer2(agg1, hw1p, dinv, b1.reshape(1, H1), W2)
    agg2 = _make_agg(H2)(hw2p, src_r, dst_r)
    return _final(agg2, hw2p, dinv, b2.reshape(1, H2),
                  batch.reshape(N, 1), fc_w, fc_b)


# speedup vs baseline: 1.2809x; 1.2809x over previous
import functools

import jax
import jax.numpy as jnp
from jax import lax
from jax.experimental import pallas as pl
from jax.experimental.pallas import tpu as pltpu
from jax.experimental.pallas import tpu_sc as plsc

N = 10000
E = 320000
F_IN = 128
H1 = 64
H2 = 32
C = 3
G = 64

NCORE = 2
NSUB = 16
K = 128
TCH = E // K
FCH = 85
SCH = 71
DCH = 78
NP = 10240
RPT = NP // NSUB


def _sc_mesh():
    return plsc.VectorSubcoreMesh(core_axis_name="c", subcore_axis_name="s")


def _zero_rows_buf(rows, h):
    def body(r, carry):
        for i in range(h // 16):
            rows[r, pl.ds(i * 16, 16)] = jnp.zeros((16,), jnp.float32)
        return carry
    lax.fori_loop(0, K, body, 0)


def _zero_acc_slice(rows, acc, base):
    nfull = RPT // K
    rem = RPT % K
    for b in range(nfull):
        pltpu.sync_copy(rows, acc.at[pl.ds(base + b * K, K)])
    if rem:
        pltpu.sync_copy(rows.at[pl.ds(0, rem)],
                        acc.at[pl.ds(base + nfull * K, rem)])


def _make_deg():
    @functools.partial(
        pl.kernel,
        out_type=jax.ShapeDtypeStruct((NCORE, NP, 16), jnp.float32),
        mesh=_sc_mesh(),
        scratch_types=[
            pltpu.VMEM((DCH + 1, K), jnp.int32),
            pltpu.VMEM((K, 16), jnp.float32),
            pltpu.VMEM_SHARED((NP, 16), jnp.float32),
            pltpu.SemaphoreType.DMA,
        ],
        compiler_params=pltpu.CompilerParams(use_tc_tiling_on_sc=False),
    )
    def deg(dst_hbm, out_hbm, didx, rows, acc, sem):
        c = lax.axis_index("c")
        s = lax.axis_index("s")
        w = s * NCORE + c
        base = s * RPT
        _zero_rows_buf(rows, 16)
        _zero_acc_slice(rows, acc, base)
        start = w * DCH + jnp.maximum(0, w - 28)
        count = DCH + jnp.where(w >= 28, 1, 0)
        pltpu.sync_copy(dst_hbm.at[pl.ds(start, DCH + 1)], didx)

        def fill_ones(r, carry):
            rows[r, :] = jnp.ones((16,), jnp.float32)
            return carry
        lax.fori_loop(0, K, fill_ones, 0)
        plsc.subcore_barrier()

        def chunk(j, carry):
            pltpu.async_copy(rows, acc.at[didx.at[j]], sem, add=True).wait()
            return carry
        lax.fori_loop(0, count, chunk, 0)
        plsc.subcore_barrier()
        pltpu.sync_copy(acc.at[pl.ds(base, RPT)],
                        out_hbm.at[c, pl.ds(base, RPT)])

    return deg


NBUF = 4


def _make_agg(h):
    @functools.partial(
        pl.kernel,
        out_type=jax.ShapeDtypeStruct((NCORE, NP, h), jnp.float32),
        mesh=_sc_mesh(),
        scratch_types=[
            pltpu.VMEM((FCH + 1, K), jnp.int32),
            pltpu.VMEM((FCH + 1, K), jnp.int32),
            [pltpu.VMEM((K, h), jnp.float32) for _ in range(NBUF)],
            pltpu.VMEM_SHARED((NP, h), jnp.float32),
            [pltpu.SemaphoreType.DMA for _ in range(NBUF)],
            pltpu.SemaphoreType.DMA,
        ],
        compiler_params=pltpu.CompilerParams(use_tc_tiling_on_sc=False),
    )
    def agg(table_hbm, src_hbm, dst_hbm, out_hbm, sidx, didx, rows, acc,
            sem_g, sem_s):
        c = lax.axis_index("c")
        s = lax.axis_index("s")
        base = s * RPT
        is_fast = c == 0
        start = jnp.where(is_fast, s * FCH,
                          16 * FCH + s * SCH + jnp.maximum(0, s - 12))
        count = jnp.where(is_fast, FCH, SCH + jnp.where(s >= 12, 1, 0))
        _zero_rows_buf(rows[0], h)
        _zero_acc_slice(rows[0], acc, base)

        @pl.when(is_fast)
        def _():
            pltpu.sync_copy(src_hbm.at[pl.ds(start, FCH)],
                            sidx.at[pl.ds(0, FCH)])
            pltpu.sync_copy(dst_hbm.at[pl.ds(start, FCH)],
                            didx.at[pl.ds(0, FCH)])

        @pl.when(jnp.logical_not(is_fast))
        def _():
            pltpu.sync_copy(src_hbm.at[pl.ds(start, SCH + 1)],
                            sidx.at[pl.ds(0, SCH + 1)])
            pltpu.sync_copy(dst_hbm.at[pl.ds(start, SCH + 1)],
                            didx.at[pl.ds(0, SCH + 1)])
        plsc.subcore_barrier()

        rounds = count // NBUF

        def round_body(i, carry):
            j0 = i * NBUF
            descs = [
                pltpu.async_copy(table_hbm.at[sidx.at[j0 + b]], rows[b],
                                 sem_g[b])
                for b in range(NBUF)
            ]
            prev = None
            for b in range(NBUF):
                descs[b].wait()
                if prev is not None:
                    prev.wait()
                prev = pltpu.async_copy(rows[b], acc.at[didx.at[j0 + b]],
                                        sem_s, add=True)
            prev.wait()
            return carry
        lax.fori_loop(0, rounds, round_body, 0)

        def tail_body(j, carry):
            pltpu.async_copy(table_hbm.at[sidx.at[j]], rows[0],
                             sem_g[0]).wait()
            pltpu.async_copy(rows[0], acc.at[didx.at[j]], sem_s,
                             add=True).wait()
            return carry
        lax.fori_loop(rounds * NBUF, count, tail_body, 0)
        plsc.subcore_barrier()
        pltpu.sync_copy(acc.at[pl.ds(base, RPT)],
                        out_hbm.at[c, pl.ds(base, RPT)])

    return agg


def _mm1(x, w1):
    def body(x_ref, w_ref, o_ref):
        o_ref[...] = jnp.dot(x_ref[...], w_ref[...],
                             preferred_element_type=jnp.float32)
    return pl.pallas_call(
        body, out_shape=jax.ShapeDtypeStruct((N, H1), jnp.float32))(x, w1)


def _scale(hw1, deg16):
    def body(hw_ref, deg_ref, hwp_ref, dinv_ref):
        deg = deg_ref[0, :N, 0] + deg_ref[1, :N, 0] + 1.0
        dinv = lax.rsqrt(deg)[:, None]
        dinv_ref[...] = dinv
        hwp_ref[...] = hw_ref[...] * dinv
    return pl.pallas_call(
        body,
        out_shape=[jax.ShapeDtypeStruct((N, H1), jnp.float32),
                   jax.ShapeDtypeStruct((N, 1), jnp.float32)])(hw1, deg16)


def _layer2(agg1, hw1p, dinv, b1, w2):
    def body(agg_ref, hwp_ref, dinv_ref, b_ref, w_ref, o_ref):
        aggsum = agg_ref[0, :N, :] + agg_ref[1, :N, :] + hwp_ref[...]
        hcur = jnp.maximum(aggsum * dinv_ref[...] + b_ref[...], 0.0)
        hw2 = jnp.dot(hcur, w_ref[...], preferred_element_type=jnp.float32)
        o_ref[...] = hw2 * dinv_ref[...]
    return pl.pallas_call(
        body, out_shape=jax.ShapeDtypeStruct((N, H2), jnp.float32))(
            agg1, hw1p, dinv, b1, w2)


def _final(agg2, hw2p, dinv, b2, batch2d, fc_w, fc_b):
    def body(agg_ref, hwp_ref, dinv_ref, b_ref, bat_ref, fw_ref, fb_ref,
             o_ref):
        aggsum = agg_ref[0, :N, :] + agg_ref[1, :N, :] + hwp_ref[...]
        hcur = jnp.maximum(aggsum * dinv_ref[...] + b_ref[...], 0.0)
        onehot = (bat_ref[...] == lax.broadcasted_iota(
            jnp.int32, (N, G), 1)).astype(jnp.float32)
        sums = lax.dot_general(onehot, hcur, (((0,), (0,)), ((), ())),
                               preferred_element_type=jnp.float32)
        cnt = jnp.sum(onehot, axis=0)[:, None]
        pooled = sums / jnp.maximum(cnt, 1.0)
        o_ref[...] = jnp.dot(pooled, fw_ref[...],
                             preferred_element_type=jnp.float32) + fb_ref[...]
    return pl.pallas_call(
        body, out_shape=jax.ShapeDtypeStruct((G, C), jnp.float32))(
            agg2, hw2p, dinv, b2, batch2d, fc_w, fc_b)


def kernel(x, edge_index, batch, W1, b1, W2, b2, fc_w, fc_b):
    src_r = edge_index[0].reshape(TCH, K)
    dst_r = edge_index[1].reshape(TCH, K)

    deg16 = _make_deg()(dst_r)
    hw1 = _mm1(x, W1)
    hw1p, dinv = _scale(hw1, deg16)
    agg1 = _make_agg(H1)(hw1p, src_r, dst_r)
    hw2p = _layer2(agg1, hw1p, dinv, b1.reshape(1, H1), W2)
    agg2 = _make_agg(H2)(hw2p, src_r, dst_r)
    return _final(agg2, hw2p, dinv, b2.reshape(1, H2),
                  batch.reshape(N, 1), fc_w, fc_b)
